# baseline (speedup 1.0000x reference)
import functools

import jax
import jax.numpy as jnp
from jax import lax
from jax.experimental import pallas as pl
from jax.experimental.pallas import tpu as pltpu
from jax.experimental.pallas import tpu_sc as plsc

B, N, C = 16, 8192, 256
S, K = 64, 8
RAD2 = 16.0

NW = 32
SPW = (B * S) // NW
RPW = SPW * K


def _fps_kernel(cx_ref, cy_ref, cz_ref, sflat_ref, scx_ref, scy_ref,
                scz_ref, dist_ref, idxf_ref, cxs_ref, cys_ref, czs_ref):
    cols = lax.broadcasted_iota(jnp.int32, (B, N), 1).astype(jnp.float32)
    dist_ref[...] = jnp.full((B, N), 1e10, dtype=jnp.float32)
    idxf_ref[...] = jnp.zeros((B, S * 128), dtype=jnp.float32)
    cxs_ref[...] = jnp.full((B, S * 128), -1e30, dtype=jnp.float32)
    cys_ref[...] = jnp.full((B, S * 128), -1e30, dtype=jnp.float32)
    czs_ref[...] = jnp.full((B, S * 128), -1e30, dtype=jnp.float32)

    def step(t, far):
        cx = cx_ref[...]
        cy = cy_ref[...]
        cz = cz_ref[...]
        sel = cols == far
        cxc = jnp.sum(jnp.where(sel, cx, 0.0), axis=1, keepdims=True)
        cyc = jnp.sum(jnp.where(sel, cy, 0.0), axis=1, keepdims=True)
        czc = jnp.sum(jnp.where(sel, cz, 0.0), axis=1, keepdims=True)
        d = (cx - cxc) ** 2 + (cy - cyc) ** 2 + (cz - czc) ** 2
        dist = jnp.minimum(dist_ref[...], d)
        dist_ref[...] = dist
        m = jnp.max(dist, axis=1, keepdims=True)
        farn = jnp.min(jnp.where(dist == m, cols, float(N)), axis=1, keepdims=True)
        slot = pl.ds(pl.multiple_of(t * 128, 128), 1)
        idxf_ref[:, slot] = far
        cxs_ref[:, slot] = cxc
        cys_ref[:, slot] = cyc
        czs_ref[:, slot] = czc
        return farn

    far0 = jnp.zeros((B, 1), dtype=jnp.float32)
    lax.fori_loop(0, S, step, far0)
    idx = jnp.max(idxf_ref[...].reshape(B, S, 128), axis=2)
    bi = lax.broadcasted_iota(jnp.int32, (B, S), 0)
    sflat_ref[...] = idx.astype(jnp.int32) + bi * N
    scx_ref[...] = jnp.max(cxs_ref[...].reshape(B, S, 128), axis=2)
    scy_ref[...] = jnp.max(cys_ref[...].reshape(B, S, 128), axis=2)
    scz_ref[...] = jnp.max(czs_ref[...].reshape(B, S, 128), axis=2)


def _bq_kernel(cx_ref, cy_ref, cz_ref, sx_ref, sy_ref, sz_ref, cm_ref,
               gidx_ref, coords_ref, gsc_ref, csc_ref):
    b = pl.program_id(0)
    cx = cx_ref[0]
    cy = cy_ref[0]
    cz = cz_ref[0]
    scx = sx_ref[0]
    scy = sy_ref[0]
    scz = sz_ref[0]
    cols = lax.broadcasted_iota(jnp.int32, (S, N), 1).astype(jnp.float32)
    nf = float(N)

    sqr = (scx - cx) ** 2 + (scy - cy) ** 2 + (scz - cz) ** 2
    mask = jnp.logical_not(sqr > RAD2)

    gsc_ref[...] = jnp.zeros((S, K * 128), dtype=jnp.float32)
    v = jnp.where(mask, cols, nf)
    g0 = None
    for k in range(K):
        gk = jnp.min(v, axis=1, keepdims=True)
        if k == 0:
            g0 = gk
        gsc_ref[:, k * 128:k * 128 + 1] = jnp.where(gk == nf, g0, gk)
        v = jnp.where(v == gk, nf, v)
    gidx8 = jnp.max(gsc_ref[...].reshape(S, K, 128), axis=2)
    gidx_ref[0] = gidx8.astype(jnp.int32) + b * N

    selmf = jnp.where(jnp.logical_and(mask, v == nf), 1.0, 0.0)
    fself = jnp.where(cols == g0, 1.0, 0.0)
    lhs = jnp.concatenate([selmf, fself], axis=0)
    res = jnp.dot(lhs, cm_ref[0], preferred_element_type=jnp.float32)
    sums = res[0:S]
    firsts = res[S:2 * S]
    nsel = sums[:, 3:4]
    csc_ref[...] = jnp.full((S, 8 * 128), -1e30, dtype=jnp.float32)
    csc_ref[:, 0:1] = scx
    csc_ref[:, 128:129] = scy
    csc_ref[:, 256:257] = scz
    csc_ref[:, 384:385] = (sums[:, 0:1] + (8.0 - nsel) * firsts[:, 0:1]) * 0.125 - scx
    csc_ref[:, 512:513] = (sums[:, 1:2] + (8.0 - nsel) * firsts[:, 1:2]) * 0.125 - scy
    csc_ref[:, 640:641] = (sums[:, 2:3] + (8.0 - nsel) * firsts[:, 2:3]) * 0.125 - scz
    coords_ref[0] = jnp.max(csc_ref[...].reshape(S, 8, 128), axis=2)


def _sc_gather_body(xf_hbm, gflat_hbm, sflat_hbm, gx_hbm, sx_hbm,
                    gidx_v, sidx_v, rows_v, srows_v, gmax_v, sem1, sem2):
    w = lax.axis_index("s") * 2 + lax.axis_index("c")
    pltpu.sync_copy(gflat_hbm.at[w], gidx_v)
    pltpu.sync_copy(sflat_hbm.at[w], sidx_v)
    cp1 = pltpu.async_copy(xf_hbm.at[gidx_v.at[0]], rows_v.at[pl.ds(0, 128)], sem1)
    cp2 = pltpu.async_copy(xf_hbm.at[gidx_v.at[1]], rows_v.at[pl.ds(128, 128)], sem1)
    cp3 = pltpu.async_copy(xf_hbm.at[sidx_v], srows_v, sem2)
    def pool_range(lo, hi):
        def per_sample(s, _):
            def per_chunk(c, _2):
                acc = rows_v[s * K, pl.ds(c * 16, 16)]
                for j in range(1, K):
                    acc = jnp.maximum(acc, rows_v[s * K + j, pl.ds(c * 16, 16)])
                gmax_v[s, pl.ds(c * 16, 16)] = acc
                return 0
            return lax.fori_loop(0, C // 16, per_chunk, 0)
        lax.fori_loop(lo, hi, per_sample, 0)

    cp1.wait()
    pool_range(0, SPW // 2)
    cp2.wait()
    pool_range(SPW // 2, SPW)
    cp3.wait()
    pltpu.sync_copy(gmax_v, gx_hbm.at[pl.ds(w * SPW, SPW)])
    pltpu.sync_copy(srows_v, sx_hbm.at[pl.ds(w * SPW, SPW)])


def _sc_gather_call(xf, gflat, sflat):
    mesh = plsc.VectorSubcoreMesh(core_axis_name="c", subcore_axis_name="s")
    fn = functools.partial(
        pl.kernel,
        out_type=[jax.ShapeDtypeStruct((B * S, C), jnp.float32),
                  jax.ShapeDtypeStruct((B * S, C), jnp.float32)],
        mesh=mesh,
        scratch_types=[
            pltpu.VMEM((2, 128), jnp.int32),
            pltpu.VMEM((SPW,), jnp.int32),
            pltpu.VMEM((RPW, C), jnp.float32),
            pltpu.VMEM((SPW, C), jnp.float32),
            pltpu.VMEM((SPW, C), jnp.float32),
            pltpu.SemaphoreType.DMA,
            pltpu.SemaphoreType.DMA,
        ],
    )(_sc_gather_body)
    return fn(xf, gflat, sflat)


def _attn_kernel(sx_ref, gx_ref, wq_ref, wk_ref, lnp_ref, vpad_ref, scpad_ref,
                 out1_ref, out2_ref):
    sx = sx_ref[0]
    gx = gx_ref[0]
    dx = gx - sx
    out1_ref[0] = sx + dx

    def ln(v, w_, b_):
        mu = jnp.mean(v, axis=1, keepdims=True)
        var = jnp.mean((v - mu) ** 2, axis=1, keepdims=True)
        return (v - mu) / jnp.sqrt(var + 1e-5) * w_ + b_

    nq = ln(dx, lnp_ref[0:1, :], lnp_ref[1:2, :])
    nk = ln(sx, lnp_ref[2:3, :], lnp_ref[3:4, :])
    nt = (((1,), (1,)), ((), ()))
    q = lax.dot_general(nq, wq_ref[...], nt, preferred_element_type=jnp.float32)
    k = lax.dot_general(nk, wk_ref[...], nt, preferred_element_type=jnp.float32)
    attn = lax.dot_general(q, k, nt, preferred_element_type=jnp.float32) * 0.0625
    attn = attn - jnp.max(attn, axis=1, keepdims=True)
    attn = jnp.exp(attn)
    attn = attn / jnp.sum(attn, axis=1, keepdims=True)
    c2 = jnp.dot(attn, vpad_ref[0], preferred_element_type=jnp.float32)
    out2_ref[0] = scpad_ref[0] + c2


def kernel(x, coor, Wq, Wk, ln_q_w, ln_q_b, ln_k_w, ln_k_b):
    cx = coor[:, :, 0]
    cy = coor[:, :, 1]
    cz = coor[:, :, 2]

    sflat_o, scxo, scyo, sczo = pl.pallas_call(
        _fps_kernel,
        out_shape=[
            jax.ShapeDtypeStruct((B, S), jnp.int32),
            jax.ShapeDtypeStruct((B, S), jnp.float32),
            jax.ShapeDtypeStruct((B, S), jnp.float32),
            jax.ShapeDtypeStruct((B, S), jnp.float32),
        ],
        scratch_shapes=[pltpu.VMEM((B, N), jnp.float32),
                        pltpu.VMEM((B, S * 128), jnp.float32),
                        pltpu.VMEM((B, S * 128), jnp.float32),
                        pltpu.VMEM((B, S * 128), jnp.float32),
                        pltpu.VMEM((B, S * 128), jnp.float32)],
    )(cx, cy, cz)

    gidxf, coords = pl.pallas_call(
        _bq_kernel,
        grid=(B,),
        in_specs=[
            pl.BlockSpec((1, 1, N), lambda b: (b, 0, 0)),
            pl.BlockSpec((1, 1, N), lambda b: (b, 0, 0)),
            pl.BlockSpec((1, 1, N), lambda b: (b, 0, 0)),
            pl.BlockSpec((1, S, 1), lambda b: (b, 0, 0)),
            pl.BlockSpec((1, S, 1), lambda b: (b, 0, 0)),
            pl.BlockSpec((1, S, 1), lambda b: (b, 0, 0)),
            pl.BlockSpec((1, N, 8), lambda b: (b, 0, 0)),
        ],
        out_specs=[
            pl.BlockSpec((1, S, K), lambda b: (b, 0, 0)),
            pl.BlockSpec((1, S, 8), lambda b: (b, 0, 0)),
        ],
        out_shape=[
            jax.ShapeDtypeStruct((B, S, K), jnp.int32),
            jax.ShapeDtypeStruct((B, S, 8), jnp.float32),
        ],
        scratch_shapes=[pltpu.VMEM((S, K * 128), jnp.float32),
                        pltpu.VMEM((S, 8 * 128), jnp.float32)],
    )(cx.reshape(B, 1, N), cy.reshape(B, 1, N), cz.reshape(B, 1, N),
      scxo.reshape(B, S, 1), scyo.reshape(B, S, 1), sczo.reshape(B, S, 1),
      jnp.concatenate([coor, jnp.ones((B, N, 1), jnp.float32),
                       jnp.zeros((B, N, 4), jnp.float32)], axis=2))

    gflat = gidxf.reshape(NW, 2, 128)
    sflat = sflat_o.reshape(NW, SPW)
    gx_flat, sx_flat = _sc_gather_call(x.reshape(B * N, C), gflat, sflat)

    sxr = sx_flat.reshape(B, S, C)
    gxr = gx_flat.reshape(B, S, C)
    sample_coor = coords[:, :, 0:3]
    diff_coor = coords[:, :, 3:6]
    v2d = jnp.transpose(diff_coor, (0, 2, 1)).reshape(B, S, 3)
    vpad = jnp.pad(v2d, ((0, 0), (0, 0), (0, 5)))
    scpad = jnp.pad(sample_coor, ((0, 0), (0, 0), (0, 5)))
    lnp = jnp.concatenate([ln_q_w.reshape(1, C), ln_q_b.reshape(1, C),
                           ln_k_w.reshape(1, C), ln_k_b.reshape(1, C)], axis=0)

    out1, out2pad = pl.pallas_call(
        _attn_kernel,
        grid=(B,),
        in_specs=[
            pl.BlockSpec((1, S, C), lambda b: (b, 0, 0)),
            pl.BlockSpec((1, S, C), lambda b: (b, 0, 0)),
            pl.BlockSpec((C, C), lambda b: (0, 0)),
            pl.BlockSpec((C, C), lambda b: (0, 0)),
            pl.BlockSpec((4, C), lambda b: (0, 0)),
            pl.BlockSpec((1, S, 8), lambda b: (b, 0, 0)),
            pl.BlockSpec((1, S, 8), lambda b: (b, 0, 0)),
        ],
        out_specs=[
            pl.BlockSpec((1, S, C), lambda b: (b, 0, 0)),
            pl.BlockSpec((1, S, 8), lambda b: (b, 0, 0)),
        ],
        out_shape=[
            jax.ShapeDtypeStruct((B, S, C), jnp.float32),
            jax.ShapeDtypeStruct((B, S, 8), jnp.float32),
        ],
    )(sxr, gxr, Wq, Wk, lnp, vpad, scpad)

    return (out1, out2pad[:, :, 0:3])


# speedup vs baseline: 1.4275x; 1.4275x over previous
import functools

import jax
import jax.numpy as jnp
from jax import lax
from jax.experimental import pallas as pl
from jax.experimental.pallas import tpu as pltpu
from jax.experimental.pallas import tpu_sc as plsc

B, N, C = 16, 8192, 256
S, K = 64, 8
RAD2 = 16.0

NW = 32
SPW = (B * S) // NW
RPW = SPW * K


def _fps_kernel(cx_ref, cy_ref, cz_ref, sflat_ref, scx_ref, scy_ref,
                scz_ref, cm_ref, dist_ref, idxf_ref, cxs_ref, cys_ref, czs_ref):
    cols = lax.broadcasted_iota(jnp.int32, (B, N), 1).astype(jnp.float32)
    dist_ref[...] = jnp.full((B, N), 1e10, dtype=jnp.float32)
    idxf_ref[...] = jnp.zeros((B, S * 128), dtype=jnp.float32)
    cxs_ref[...] = jnp.full((B, S * 128), -1e30, dtype=jnp.float32)
    cys_ref[...] = jnp.full((B, S * 128), -1e30, dtype=jnp.float32)
    czs_ref[...] = jnp.full((B, S * 128), -1e30, dtype=jnp.float32)

    def step(t, far):
        cx = cx_ref[...]
        cy = cy_ref[...]
        cz = cz_ref[...]
        sel = cols == far
        cxc = jnp.sum(jnp.where(sel, cx, 0.0), axis=1, keepdims=True)
        cyc = jnp.sum(jnp.where(sel, cy, 0.0), axis=1, keepdims=True)
        czc = jnp.sum(jnp.where(sel, cz, 0.0), axis=1, keepdims=True)
        d = (cx - cxc) ** 2 + (cy - cyc) ** 2 + (cz - czc) ** 2
        dist = jnp.minimum(dist_ref[...], d)
        dist_ref[...] = dist
        m = jnp.max(dist, axis=1, keepdims=True)
        farn = jnp.min(jnp.where(dist == m, cols, float(N)), axis=1, keepdims=True)
        slot = pl.ds(pl.multiple_of(t * 128, 128), 1)
        idxf_ref[:, slot] = far
        cxs_ref[:, slot] = cxc
        cys_ref[:, slot] = cyc
        czs_ref[:, slot] = czc
        return farn

    far0 = jnp.zeros((B, 1), dtype=jnp.float32)
    lax.fori_loop(0, S, step, far0)
    idx = jnp.max(idxf_ref[...].reshape(B, S, 128), axis=2)
    bi = lax.broadcasted_iota(jnp.int32, (B, S), 0)
    sflat_ref[...] = idx.astype(jnp.int32) + bi * N
    scx_ref[...] = jnp.max(cxs_ref[...].reshape(B, S, 128), axis=2)
    scy_ref[...] = jnp.max(cys_ref[...].reshape(B, S, 128), axis=2)
    scz_ref[...] = jnp.max(czs_ref[...].reshape(B, S, 128), axis=2)
    cm_ref[:, 0, :] = cx_ref[...]
    cm_ref[:, 1, :] = cy_ref[...]
    cm_ref[:, 2, :] = cz_ref[...]
    cm_ref[:, 3, :] = jnp.ones((B, N), dtype=jnp.float32)


def _bq_kernel(cx_ref, cy_ref, cz_ref, sx_ref, sy_ref, sz_ref, cm_ref,
               gidx_ref, coords_ref, gsc_ref, csc_ref):
    b = pl.program_id(0)
    cx = cx_ref[0]
    cy = cy_ref[0]
    cz = cz_ref[0]
    scx = sx_ref[0]
    scy = sy_ref[0]
    scz = sz_ref[0]
    cols = lax.broadcasted_iota(jnp.int32, (S, N), 1).astype(jnp.float32)
    nf = float(N)

    sqr = (scx - cx) ** 2 + (scy - cy) ** 2 + (scz - cz) ** 2
    mask = jnp.logical_not(sqr > RAD2)

    gsc_ref[...] = jnp.zeros((S, K * 128), dtype=jnp.float32)
    v = jnp.where(mask, cols, nf)
    g0 = None
    for k in range(K):
        gk = jnp.min(v, axis=1, keepdims=True)
        if k == 0:
            g0 = gk
        gsc_ref[:, k * 128:k * 128 + 1] = jnp.where(gk == nf, g0, gk)
        v = jnp.where(v == gk, nf, v)
    gidx8 = jnp.max(gsc_ref[...].reshape(S, K, 128), axis=2)
    gidx_ref[0] = gidx8.astype(jnp.int32) + b * N

    selmf = jnp.where(jnp.logical_and(mask, v == nf), 1.0, 0.0)
    fself = jnp.where(cols == g0, 1.0, 0.0)
    lhs = jnp.concatenate([selmf, fself], axis=0)
    res = lax.dot_general(lhs, cm_ref[0], (((1,), (1,)), ((), ())),
                          preferred_element_type=jnp.float32)
    sums = res[0:S]
    firsts = res[S:2 * S]
    nsel = sums[:, 3:4]
    csc_ref[...] = jnp.full((S, 8 * 128), -1e30, dtype=jnp.float32)
    csc_ref[:, 0:1] = scx
    csc_ref[:, 128:129] = scy
    csc_ref[:, 256:257] = scz
    csc_ref[:, 384:385] = (sums[:, 0:1] + (8.0 - nsel) * firsts[:, 0:1]) * 0.125 - scx
    csc_ref[:, 512:513] = (sums[:, 1:2] + (8.0 - nsel) * firsts[:, 1:2]) * 0.125 - scy
    csc_ref[:, 640:641] = (sums[:, 2:3] + (8.0 - nsel) * firsts[:, 2:3]) * 0.125 - scz
    coords_ref[0] = jnp.max(csc_ref[...].reshape(S, 8, 128), axis=2)


def _sc_gather_body(xf_hbm, gflat_hbm, sflat_hbm, gx_hbm, sx_hbm,
                    gidx_v, sidx_v, rows_v, srows_v, gmax_v, sem1, sem2):
    w = lax.axis_index("s") * 2 + lax.axis_index("c")
    pltpu.sync_copy(gflat_hbm.at[w], gidx_v)
    pltpu.sync_copy(sflat_hbm.at[w], sidx_v)
    cp1 = pltpu.async_copy(xf_hbm.at[gidx_v.at[0]], rows_v.at[pl.ds(0, 128)], sem1)
    cp2 = pltpu.async_copy(xf_hbm.at[gidx_v.at[1]], rows_v.at[pl.ds(128, 128)], sem1)
    cp3 = pltpu.async_copy(xf_hbm.at[sidx_v], srows_v, sem2)
    def pool_range(lo, hi):
        def per_sample(s, _):
            def per_chunk(c, _2):
                acc = rows_v[s * K, pl.ds(c * 16, 16)]
                for j in range(1, K):
                    acc = jnp.maximum(acc, rows_v[s * K + j, pl.ds(c * 16, 16)])
                gmax_v[s, pl.ds(c * 16, 16)] = acc
                return 0
            return lax.fori_loop(0, C // 16, per_chunk, 0)
        lax.fori_loop(lo, hi, per_sample, 0)

    cp1.wait()
    pool_range(0, SPW // 2)
    cp2.wait()
    pool_range(SPW // 2, SPW)
    cp3.wait()
    pltpu.sync_copy(gmax_v, gx_hbm.at[pl.ds(w * SPW, SPW)])
    pltpu.sync_copy(srows_v, sx_hbm.at[pl.ds(w * SPW, SPW)])


def _sc_gather_call(xf, gflat, sflat):
    mesh = plsc.VectorSubcoreMesh(core_axis_name="c", subcore_axis_name="s")
    fn = functools.partial(
        pl.kernel,
        out_type=[jax.ShapeDtypeStruct((B * S, C), jnp.float32),
                  jax.ShapeDtypeStruct((B * S, C), jnp.float32)],
        mesh=mesh,
        scratch_types=[
            pltpu.VMEM((2, 128), jnp.int32),
            pltpu.VMEM((SPW,), jnp.int32),
            pltpu.VMEM((RPW, C), jnp.float32),
            pltpu.VMEM((SPW, C), jnp.float32),
            pltpu.VMEM((SPW, C), jnp.float32),
            pltpu.SemaphoreType.DMA,
            pltpu.SemaphoreType.DMA,
        ],
    )(_sc_gather_body)
    return fn(xf, gflat, sflat)


def _attn_kernel(sx_ref, gx_ref, wq_ref, wk_ref, lnp_ref, vpad_ref, scpad_ref,
                 out1_ref, out2_ref):
    sx = sx_ref[0]
    gx = gx_ref[0]
    dx = gx - sx
    out1_ref[0] = sx + dx

    def ln(v, w_, b_):
        mu = jnp.mean(v, axis=1, keepdims=True)
        var = jnp.mean((v - mu) ** 2, axis=1, keepdims=True)
        return (v - mu) / jnp.sqrt(var + 1e-5) * w_ + b_

    nq = ln(dx, lnp_ref[0:1, :], lnp_ref[1:2, :])
    nk = ln(sx, lnp_ref[2:3, :], lnp_ref[3:4, :])
    nt = (((1,), (1,)), ((), ()))
    q = lax.dot_general(nq, wq_ref[...], nt, preferred_element_type=jnp.float32)
    k = lax.dot_general(nk, wk_ref[...], nt, preferred_element_type=jnp.float32)
    attn = lax.dot_general(q, k, nt, preferred_element_type=jnp.float32) * 0.0625
    attn = attn - jnp.max(attn, axis=1, keepdims=True)
    attn = jnp.exp(attn)
    attn = attn / jnp.sum(attn, axis=1, keepdims=True)
    c2 = jnp.dot(attn, vpad_ref[0], preferred_element_type=jnp.float32)
    out2_ref[0] = scpad_ref[0] + c2


def kernel(x, coor, Wq, Wk, ln_q_w, ln_q_b, ln_k_w, ln_k_b):
    cx = coor[:, :, 0]
    cy = coor[:, :, 1]
    cz = coor[:, :, 2]

    sflat_o, scxo, scyo, sczo, cm8 = pl.pallas_call(
        _fps_kernel,
        out_shape=[
            jax.ShapeDtypeStruct((B, S), jnp.int32),
            jax.ShapeDtypeStruct((B, S), jnp.float32),
            jax.ShapeDtypeStruct((B, S), jnp.float32),
            jax.ShapeDtypeStruct((B, S), jnp.float32),
            jax.ShapeDtypeStruct((B, 4, N), jnp.float32),
        ],
        scratch_shapes=[pltpu.VMEM((B, N), jnp.float32),
                        pltpu.VMEM((B, S * 128), jnp.float32),
                        pltpu.VMEM((B, S * 128), jnp.float32),
                        pltpu.VMEM((B, S * 128), jnp.float32),
                        pltpu.VMEM((B, S * 128), jnp.float32)],
    )(cx, cy, cz)

    gidxf, coords = pl.pallas_call(
        _bq_kernel,
        grid=(B,),
        in_specs=[
            pl.BlockSpec((1, 1, N), lambda b: (b, 0, 0)),
            pl.BlockSpec((1, 1, N), lambda b: (b, 0, 0)),
            pl.BlockSpec((1, 1, N), lambda b: (b, 0, 0)),
            pl.BlockSpec((1, S, 1), lambda b: (b, 0, 0)),
            pl.BlockSpec((1, S, 1), lambda b: (b, 0, 0)),
            pl.BlockSpec((1, S, 1), lambda b: (b, 0, 0)),
            pl.BlockSpec((1, 4, N), lambda b: (b, 0, 0)),
        ],
        out_specs=[
            pl.BlockSpec((1, S, K), lambda b: (b, 0, 0)),
            pl.BlockSpec((1, S, 8), lambda b: (b, 0, 0)),
        ],
        out_shape=[
            jax.ShapeDtypeStruct((B, S, K), jnp.int32),
            jax.ShapeDtypeStruct((B, S, 8), jnp.float32),
        ],
        scratch_shapes=[pltpu.VMEM((S, K * 128), jnp.float32),
                        pltpu.VMEM((S, 8 * 128), jnp.float32)],
    )(cx.reshape(B, 1, N), cy.reshape(B, 1, N), cz.reshape(B, 1, N),
      scxo.reshape(B, S, 1), scyo.reshape(B, S, 1), sczo.reshape(B, S, 1),
      cm8)

    gflat = gidxf.reshape(NW, 2, 128)
    sflat = sflat_o.reshape(NW, SPW)
    gx_flat, sx_flat = _sc_gather_call(x.reshape(B * N, C), gflat, sflat)

    sxr = sx_flat.reshape(B, S, C)
    gxr = gx_flat.reshape(B, S, C)
    sample_coor = coords[:, :, 0:3]
    diff_coor = coords[:, :, 3:6]
    v2d = jnp.transpose(diff_coor, (0, 2, 1)).reshape(B, S, 3)
    vpad = jnp.pad(v2d, ((0, 0), (0, 0), (0, 5)))
    scpad = jnp.pad(sample_coor, ((0, 0), (0, 0), (0, 5)))
    lnp = jnp.concatenate([ln_q_w.reshape(1, C), ln_q_b.reshape(1, C),
                           ln_k_w.reshape(1, C), ln_k_b.reshape(1, C)], axis=0)

    out1, out2pad = pl.pallas_call(
        _attn_kernel,
        grid=(B,),
        in_specs=[
            pl.BlockSpec((1, S, C), lambda b: (b, 0, 0)),
            pl.BlockSpec((1, S, C), lambda b: (b, 0, 0)),
            pl.BlockSpec((C, C), lambda b: (0, 0)),
            pl.BlockSpec((C, C), lambda b: (0, 0)),
            pl.BlockSpec((4, C), lambda b: (0, 0)),
            pl.BlockSpec((1, S, 8), lambda b: (b, 0, 0)),
            pl.BlockSpec((1, S, 8), lambda b: (b, 0, 0)),
        ],
        out_specs=[
            pl.BlockSpec((1, S, C), lambda b: (b, 0, 0)),
            pl.BlockSpec((1, S, 8), lambda b: (b, 0, 0)),
        ],
        out_shape=[
            jax.ShapeDtypeStruct((B, S, C), jnp.float32),
            jax.ShapeDtypeStruct((B, S, 8), jnp.float32),
        ],
    )(sxr, gxr, Wq, Wk, lnp, vpad, scpad)

    return (out1, out2pad[:, :, 0:3])


# speedup vs baseline: 1.5085x; 1.0567x over previous
import functools

import jax
import jax.numpy as jnp
from jax import lax
from jax.experimental import pallas as pl
from jax.experimental.pallas import tpu as pltpu
from jax.experimental.pallas import tpu_sc as plsc

B, N, C = 16, 8192, 256
S, K = 64, 8
RAD2 = 16.0

NW = 32
SPW = (B * S) // NW
RPW = SPW * K


def _fps_kernel(cx_ref, cy_ref, cz_ref, sflat_ref, scx_ref, scy_ref,
                scz_ref, cm_ref, dist_ref, idxf_ref, cxs_ref, cys_ref, czs_ref):
    cols = lax.broadcasted_iota(jnp.int32, (B, N), 1).astype(jnp.float32)
    dist_ref[...] = jnp.full((B, N), 1e10, dtype=jnp.float32)
    idxf_ref[...] = jnp.zeros((B, S * 128), dtype=jnp.float32)
    cxs_ref[...] = jnp.full((B, S * 128), -1e30, dtype=jnp.float32)
    cys_ref[...] = jnp.full((B, S * 128), -1e30, dtype=jnp.float32)
    czs_ref[...] = jnp.full((B, S * 128), -1e30, dtype=jnp.float32)

    def step(t, far):
        cx = cx_ref[...]
        cy = cy_ref[...]
        cz = cz_ref[...]
        sel = cols == far
        cxc = jnp.sum(jnp.where(sel, cx, 0.0), axis=1, keepdims=True)
        cyc = jnp.sum(jnp.where(sel, cy, 0.0), axis=1, keepdims=True)
        czc = jnp.sum(jnp.where(sel, cz, 0.0), axis=1, keepdims=True)
        d = (cx - cxc) ** 2 + (cy - cyc) ** 2 + (cz - czc) ** 2
        dist = jnp.minimum(dist_ref[...], d)
        dist_ref[...] = dist
        m = jnp.max(dist, axis=1, keepdims=True)
        farn = jnp.min(jnp.where(dist == m, cols, float(N)), axis=1, keepdims=True)
        slot = pl.ds(pl.multiple_of(t * 128, 128), 1)
        idxf_ref[:, slot] = far
        cxs_ref[:, slot] = cxc
        cys_ref[:, slot] = cyc
        czs_ref[:, slot] = czc
        return farn

    far0 = jnp.zeros((B, 1), dtype=jnp.float32)
    lax.fori_loop(0, S, step, far0)
    idx = jnp.max(idxf_ref[...].reshape(B, S, 128), axis=2)
    bi = lax.broadcasted_iota(jnp.int32, (B, S), 0)
    sflat_ref[...] = idx.astype(jnp.int32) + bi * N
    scx_ref[...] = jnp.max(cxs_ref[...].reshape(B, S, 128), axis=2)
    scy_ref[...] = jnp.max(cys_ref[...].reshape(B, S, 128), axis=2)
    scz_ref[...] = jnp.max(czs_ref[...].reshape(B, S, 128), axis=2)
    cm_ref[:, 0, :] = cx_ref[...]
    cm_ref[:, 1, :] = cy_ref[...]
    cm_ref[:, 2, :] = cz_ref[...]
    cm_ref[:, 3, :] = jnp.ones((B, N), dtype=jnp.float32)


BPP = 4


def _bq_kernel(cx_ref, cy_ref, cz_ref, sx_ref, sy_ref, sz_ref, cm_ref,
               gidx_ref, coords_ref, gsc_ref, csc_ref):
    cols = lax.broadcasted_iota(jnp.int32, (S, N), 1).astype(jnp.float32)
    nf = float(N)
    for i in range(BPP):
        b = pl.program_id(0) * BPP + i
        cx = cx_ref[i]
        cy = cy_ref[i]
        cz = cz_ref[i]
        scx = sx_ref[i]
        scy = sy_ref[i]
        scz = sz_ref[i]

        sqr = (scx - cx) ** 2 + (scy - cy) ** 2 + (scz - cz) ** 2
        mask = jnp.logical_not(sqr > RAD2)

        gsc_ref[...] = jnp.zeros((S, K * 128), dtype=jnp.float32)
        v = jnp.where(mask, cols, nf)
        g0 = None
        for k in range(K):
            gk = jnp.min(v, axis=1, keepdims=True)
            if k == 0:
                g0 = gk
            gsc_ref[:, k * 128:k * 128 + 1] = jnp.where(gk == nf, g0, gk)
            v = jnp.where(v == gk, nf, v)
        gidx8 = jnp.max(gsc_ref[...].reshape(S, K, 128), axis=2)
        gidx_ref[i] = gidx8.astype(jnp.int32) + b * N

        selmf = jnp.where(jnp.logical_and(mask, v == nf), 1.0, 0.0)
        fself = jnp.where(cols == g0, 1.0, 0.0)
        lhs = jnp.concatenate([selmf, fself], axis=0)
        res = lax.dot_general(lhs, cm_ref[i], (((1,), (1,)), ((), ())),
                              preferred_element_type=jnp.float32)
        sums = res[0:S]
        firsts = res[S:2 * S]
        nsel = sums[:, 3:4]
        csc_ref[...] = jnp.full((S, 8 * 128), -1e30, dtype=jnp.float32)
        csc_ref[:, 0:1] = scx
        csc_ref[:, 128:129] = scy
        csc_ref[:, 256:257] = scz
        csc_ref[:, 384:385] = (sums[:, 0:1] + (8.0 - nsel) * firsts[:, 0:1]) * 0.125 - scx
        csc_ref[:, 512:513] = (sums[:, 1:2] + (8.0 - nsel) * firsts[:, 1:2]) * 0.125 - scy
        csc_ref[:, 640:641] = (sums[:, 2:3] + (8.0 - nsel) * firsts[:, 2:3]) * 0.125 - scz
        coords_ref[i] = jnp.max(csc_ref[...].reshape(S, 8, 128), axis=2)


def _sc_gather_body(xf_hbm, gflat_hbm, sflat_hbm, gx_hbm, sx_hbm,
                    gidx_v, sidx_v, rows_v, srows_v, gmax_v, sem1, sem2):
    w = lax.axis_index("s") * 2 + lax.axis_index("c")
    pltpu.sync_copy(gflat_hbm.at[w], gidx_v)
    pltpu.sync_copy(sflat_hbm.at[w], sidx_v)
    cp1 = pltpu.async_copy(xf_hbm.at[gidx_v.at[0]], rows_v.at[pl.ds(0, 128)], sem1)
    cp2 = pltpu.async_copy(xf_hbm.at[gidx_v.at[1]], rows_v.at[pl.ds(128, 128)], sem1)
    cp3 = pltpu.async_copy(xf_hbm.at[sidx_v], srows_v, sem2)
    def pool_range(lo, hi):
        def per_sample(s, _):
            def per_chunk(c, _2):
                acc = rows_v[s * K, pl.ds(c * 16, 16)]
                for j in range(1, K):
                    acc = jnp.maximum(acc, rows_v[s * K + j, pl.ds(c * 16, 16)])
                gmax_v[s, pl.ds(c * 16, 16)] = acc
                return 0
            return lax.fori_loop(0, C // 16, per_chunk, 0)
        lax.fori_loop(lo, hi, per_sample, 0)

    cp1.wait()
    pool_range(0, SPW // 2)
    cp2.wait()
    pool_range(SPW // 2, SPW)
    cp3.wait()
    pltpu.sync_copy(gmax_v, gx_hbm.at[pl.ds(w * SPW, SPW)])
    pltpu.sync_copy(srows_v, sx_hbm.at[pl.ds(w * SPW, SPW)])


def _sc_gather_call(xf, gflat, sflat):
    mesh = plsc.VectorSubcoreMesh(core_axis_name="c", subcore_axis_name="s")
    fn = functools.partial(
        pl.kernel,
        out_type=[jax.ShapeDtypeStruct((B * S, C), jnp.float32),
                  jax.ShapeDtypeStruct((B * S, C), jnp.float32)],
        mesh=mesh,
        scratch_types=[
            pltpu.VMEM((2, 128), jnp.int32),
            pltpu.VMEM((SPW,), jnp.int32),
            pltpu.VMEM((RPW, C), jnp.float32),
            pltpu.VMEM((SPW, C), jnp.float32),
            pltpu.VMEM((SPW, C), jnp.float32),
            pltpu.SemaphoreType.DMA,
            pltpu.SemaphoreType.DMA,
        ],
    )(_sc_gather_body)
    return fn(xf, gflat, sflat)


def _attn_kernel(sx_ref, gx_ref, wq_ref, wk_ref, lnp_ref, vpad_ref, scpad_ref,
                 out1_ref, out2_ref):
    sx = sx_ref[0]
    gx = gx_ref[0]
    dx = gx - sx
    out1_ref[0] = sx + dx

    def ln(v, w_, b_):
        mu = jnp.mean(v, axis=1, keepdims=True)
        var = jnp.mean((v - mu) ** 2, axis=1, keepdims=True)
        return (v - mu) / jnp.sqrt(var + 1e-5) * w_ + b_

    nq = ln(dx, lnp_ref[0:1, :], lnp_ref[1:2, :])
    nk = ln(sx, lnp_ref[2:3, :], lnp_ref[3:4, :])
    nt = (((1,), (1,)), ((), ()))
    q = lax.dot_general(nq, wq_ref[...], nt, preferred_element_type=jnp.float32)
    k = lax.dot_general(nk, wk_ref[...], nt, preferred_element_type=jnp.float32)
    attn = lax.dot_general(q, k, nt, preferred_element_type=jnp.float32) * 0.0625
    attn = attn - jnp.max(attn, axis=1, keepdims=True)
    attn = jnp.exp(attn)
    attn = attn / jnp.sum(attn, axis=1, keepdims=True)
    c2 = jnp.dot(attn, vpad_ref[0], preferred_element_type=jnp.float32)
    out2_ref[0] = scpad_ref[0] + c2


def kernel(x, coor, Wq, Wk, ln_q_w, ln_q_b, ln_k_w, ln_k_b):
    cx = coor[:, :, 0]
    cy = coor[:, :, 1]
    cz = coor[:, :, 2]

    sflat_o, scxo, scyo, sczo, cm8 = pl.pallas_call(
        _fps_kernel,
        out_shape=[
            jax.ShapeDtypeStruct((B, S), jnp.int32),
            jax.ShapeDtypeStruct((B, S), jnp.float32),
            jax.ShapeDtypeStruct((B, S), jnp.float32),
            jax.ShapeDtypeStruct((B, S), jnp.float32),
            jax.ShapeDtypeStruct((B, 4, N), jnp.float32),
        ],
        scratch_shapes=[pltpu.VMEM((B, N), jnp.float32),
                        pltpu.VMEM((B, S * 128), jnp.float32),
                        pltpu.VMEM((B, S * 128), jnp.float32),
                        pltpu.VMEM((B, S * 128), jnp.float32),
                        pltpu.VMEM((B, S * 128), jnp.float32)],
    )(cx, cy, cz)

    gidxf, coords = pl.pallas_call(
        _bq_kernel,
        grid=(B // BPP,),
        in_specs=[
            pl.BlockSpec((BPP, 1, N), lambda b: (b, 0, 0)),
            pl.BlockSpec((BPP, 1, N), lambda b: (b, 0, 0)),
            pl.BlockSpec((BPP, 1, N), lambda b: (b, 0, 0)),
            pl.BlockSpec((BPP, S, 1), lambda b: (b, 0, 0)),
            pl.BlockSpec((BPP, S, 1), lambda b: (b, 0, 0)),
            pl.BlockSpec((BPP, S, 1), lambda b: (b, 0, 0)),
            pl.BlockSpec((BPP, 4, N), lambda b: (b, 0, 0)),
        ],
        out_specs=[
            pl.BlockSpec((BPP, S, K), lambda b: (b, 0, 0)),
            pl.BlockSpec((BPP, S, 8), lambda b: (b, 0, 0)),
        ],
        out_shape=[
            jax.ShapeDtypeStruct((B, S, K), jnp.int32),
            jax.ShapeDtypeStruct((B, S, 8), jnp.float32),
        ],
        scratch_shapes=[pltpu.VMEM((S, K * 128), jnp.float32),
                        pltpu.VMEM((S, 8 * 128), jnp.float32)],
    )(cx.reshape(B, 1, N), cy.reshape(B, 1, N), cz.reshape(B, 1, N),
      scxo.reshape(B, S, 1), scyo.reshape(B, S, 1), sczo.reshape(B, S, 1),
      cm8)

    gflat = gidxf.reshape(NW, 2, 128)
    sflat = sflat_o.reshape(NW, SPW)
    gx_flat, sx_flat = _sc_gather_call(x.reshape(B * N, C), gflat, sflat)

    sxr = sx_flat.reshape(B, S, C)
    gxr = gx_flat.reshape(B, S, C)
    sample_coor = coords[:, :, 0:3]
    diff_coor = coords[:, :, 3:6]
    v2d = jnp.transpose(diff_coor, (0, 2, 1)).reshape(B, S, 3)
    vpad = jnp.pad(v2d, ((0, 0), (0, 0), (0, 5)))
    scpad = jnp.pad(sample_coor, ((0, 0), (0, 0), (0, 5)))
    lnp = jnp.concatenate([ln_q_w.reshape(1, C), ln_q_b.reshape(1, C),
                           ln_k_w.reshape(1, C), ln_k_b.reshape(1, C)], axis=0)

    out1, out2pad = pl.pallas_call(
        _attn_kernel,
        grid=(B,),
        in_specs=[
            pl.BlockSpec((1, S, C), lambda b: (b, 0, 0)),
            pl.BlockSpec((1, S, C), lambda b: (b, 0, 0)),
            pl.BlockSpec((C, C), lambda b: (0, 0)),
            pl.BlockSpec((C, C), lambda b: (0, 0)),
            pl.BlockSpec((4, C), lambda b: (0, 0)),
            pl.BlockSpec((1, S, 8), lambda b: (b, 0, 0)),
            pl.BlockSpec((1, S, 8), lambda b: (b, 0, 0)),
        ],
        out_specs=[
            pl.BlockSpec((1, S, C), lambda b: (b, 0, 0)),
            pl.BlockSpec((1, S, 8), lambda b: (b, 0, 0)),
        ],
        out_shape=[
            jax.ShapeDtypeStruct((B, S, C), jnp.float32),
            jax.ShapeDtypeStruct((B, S, 8), jnp.float32),
        ],
    )(sxr, gxr, Wq, Wk, lnp, vpad, scpad)

    return (out1, out2pad[:, :, 0:3])


# speedup vs baseline: 1.5133x; 1.0032x over previous
import functools

import jax
import jax.numpy as jnp
from jax import lax
from jax.experimental import pallas as pl
from jax.experimental.pallas import tpu as pltpu
from jax.experimental.pallas import tpu_sc as plsc

B, N, C = 16, 8192, 256
S, K = 64, 8
RAD2 = 16.0

NW = 32
SPW = (B * S) // NW
RPW = SPW * K


def _fps_kernel(cx_ref, cy_ref, cz_ref, sflat_ref, scx_ref, scy_ref,
                scz_ref, cm_ref, dist_ref, idxf_ref, cxs_ref, cys_ref, czs_ref):
    cols = lax.broadcasted_iota(jnp.int32, (B, N), 1).astype(jnp.float32)
    dist_ref[...] = jnp.full((B, N), 1e10, dtype=jnp.float32)
    idxf_ref[...] = jnp.zeros((B, S * 128), dtype=jnp.float32)
    cxs_ref[...] = jnp.full((B, S * 128), -1e30, dtype=jnp.float32)
    cys_ref[...] = jnp.full((B, S * 128), -1e30, dtype=jnp.float32)
    czs_ref[...] = jnp.full((B, S * 128), -1e30, dtype=jnp.float32)

    def step(t, far):
        cx = cx_ref[...]
        cy = cy_ref[...]
        cz = cz_ref[...]
        sel = cols == far
        cxc = jnp.sum(jnp.where(sel, cx, 0.0), axis=1, keepdims=True)
        cyc = jnp.sum(jnp.where(sel, cy, 0.0), axis=1, keepdims=True)
        czc = jnp.sum(jnp.where(sel, cz, 0.0), axis=1, keepdims=True)
        d = (cx - cxc) ** 2 + (cy - cyc) ** 2 + (cz - czc) ** 2
        dist = jnp.minimum(dist_ref[...], d)
        dist_ref[...] = dist
        m2 = jnp.max(dist.reshape(B, S, 128), axis=2)
        m = jnp.max(m2, axis=1, keepdims=True)
        farn = jnp.min(jnp.where(dist == m, cols, float(N)), axis=1, keepdims=True)
        slot = pl.ds(pl.multiple_of(t * 128, 128), 1)
        idxf_ref[:, slot] = far
        cxs_ref[:, slot] = cxc
        cys_ref[:, slot] = cyc
        czs_ref[:, slot] = czc
        return farn

    far0 = jnp.zeros((B, 1), dtype=jnp.float32)
    lax.fori_loop(0, S, step, far0)
    idx = jnp.max(idxf_ref[...].reshape(B, S, 128), axis=2)
    bi = lax.broadcasted_iota(jnp.int32, (B, S), 0)
    sflat_ref[...] = idx.astype(jnp.int32) + bi * N
    scx_ref[...] = jnp.max(cxs_ref[...].reshape(B, S, 128), axis=2)
    scy_ref[...] = jnp.max(cys_ref[...].reshape(B, S, 128), axis=2)
    scz_ref[...] = jnp.max(czs_ref[...].reshape(B, S, 128), axis=2)
    cm_ref[:, 0, :] = cx_ref[...]
    cm_ref[:, 1, :] = cy_ref[...]
    cm_ref[:, 2, :] = cz_ref[...]
    cm_ref[:, 3, :] = jnp.ones((B, N), dtype=jnp.float32)


BPP = 4


def _bq_kernel(cx_ref, cy_ref, cz_ref, sx_ref, sy_ref, sz_ref, cm_ref,
               gidx_ref, coords_ref, gsc_ref, csc_ref):
    cols = lax.broadcasted_iota(jnp.int32, (S, N), 1).astype(jnp.float32)
    nf = float(N)
    for i in range(BPP):
        b = pl.program_id(0) * BPP + i
        cx = cx_ref[i]
        cy = cy_ref[i]
        cz = cz_ref[i]
        scx = sx_ref[i]
        scy = sy_ref[i]
        scz = sz_ref[i]

        sqr = (scx - cx) ** 2 + (scy - cy) ** 2 + (scz - cz) ** 2
        mask = jnp.logical_not(sqr > RAD2)

        gsc_ref[...] = jnp.zeros((S, K * 128), dtype=jnp.float32)
        v = jnp.where(mask, cols, nf)
        g0 = None
        for k in range(K):
            gk = jnp.min(v, axis=1, keepdims=True)
            if k == 0:
                g0 = gk
            gsc_ref[:, k * 128:k * 128 + 1] = jnp.where(gk == nf, g0, gk)
            v = jnp.where(v == gk, nf, v)
        gidx8 = jnp.max(gsc_ref[...].reshape(S, K, 128), axis=2)
        gidx_ref[i] = gidx8.astype(jnp.int32) + b * N

        selmf = jnp.where(jnp.logical_and(mask, v == nf), 1.0, 0.0)
        fself = jnp.where(cols == g0, 1.0, 0.0)
        lhs = jnp.concatenate([selmf, fself], axis=0)
        res = lax.dot_general(lhs, cm_ref[i], (((1,), (1,)), ((), ())),
                              preferred_element_type=jnp.float32)
        sums = res[0:S]
        firsts = res[S:2 * S]
        nsel = sums[:, 3:4]
        csc_ref[...] = jnp.full((S, 8 * 128), -1e30, dtype=jnp.float32)
        csc_ref[:, 0:1] = scx
        csc_ref[:, 128:129] = scy
        csc_ref[:, 256:257] = scz
        csc_ref[:, 384:385] = (sums[:, 0:1] + (8.0 - nsel) * firsts[:, 0:1]) * 0.125 - scx
        csc_ref[:, 512:513] = (sums[:, 1:2] + (8.0 - nsel) * firsts[:, 1:2]) * 0.125 - scy
        csc_ref[:, 640:641] = (sums[:, 2:3] + (8.0 - nsel) * firsts[:, 2:3]) * 0.125 - scz
        coords_ref[i] = jnp.max(csc_ref[...].reshape(S, 8, 128), axis=2)


def _sc_gather_body(xf_hbm, gflat_hbm, sflat_hbm, gx_hbm, sx_hbm,
                    gidx_v, sidx_v, rows_v, srows_v, gmax_v, sem1, sem2):
    w = lax.axis_index("s") * 2 + lax.axis_index("c")
    pltpu.sync_copy(gflat_hbm.at[w], gidx_v)
    pltpu.sync_copy(sflat_hbm.at[w], sidx_v)
    cp1 = pltpu.async_copy(xf_hbm.at[gidx_v.at[0]], rows_v.at[pl.ds(0, 128)], sem1)
    cp2 = pltpu.async_copy(xf_hbm.at[gidx_v.at[1]], rows_v.at[pl.ds(128, 128)], sem1)
    cp3 = pltpu.async_copy(xf_hbm.at[sidx_v], srows_v, sem2)
    def pool_range(lo, hi):
        def per_sample(s, _):
            def per_chunk(c, _2):
                acc = rows_v[s * K, pl.ds(c * 16, 16)]
                for j in range(1, K):
                    acc = jnp.maximum(acc, rows_v[s * K + j, pl.ds(c * 16, 16)])
                gmax_v[s, pl.ds(c * 16, 16)] = acc
                return 0
            return lax.fori_loop(0, C // 16, per_chunk, 0)
        lax.fori_loop(lo, hi, per_sample, 0)

    cp1.wait()
    pool_range(0, SPW // 2)
    cp2.wait()
    pool_range(SPW // 2, SPW)
    cp3.wait()
    pltpu.sync_copy(gmax_v, gx_hbm.at[pl.ds(w * SPW, SPW)])
    pltpu.sync_copy(srows_v, sx_hbm.at[pl.ds(w * SPW, SPW)])


def _sc_gather_call(xf, gflat, sflat):
    mesh = plsc.VectorSubcoreMesh(core_axis_name="c", subcore_axis_name="s")
    fn = functools.partial(
        pl.kernel,
        out_type=[jax.ShapeDtypeStruct((B * S, C), jnp.float32),
                  jax.ShapeDtypeStruct((B * S, C), jnp.float32)],
        mesh=mesh,
        scratch_types=[
            pltpu.VMEM((2, 128), jnp.int32),
            pltpu.VMEM((SPW,), jnp.int32),
            pltpu.VMEM((RPW, C), jnp.float32),
            pltpu.VMEM((SPW, C), jnp.float32),
            pltpu.VMEM((SPW, C), jnp.float32),
            pltpu.SemaphoreType.DMA,
            pltpu.SemaphoreType.DMA,
        ],
    )(_sc_gather_body)
    return fn(xf, gflat, sflat)


def _attn_kernel(sx_ref, gx_ref, wq_ref, wk_ref, lnp_ref, vpad_ref, scpad_ref,
                 out1_ref, out2_ref):
    def ln(v, w_, b_):
        mu = jnp.mean(v, axis=1, keepdims=True)
        var = jnp.mean((v - mu) ** 2, axis=1, keepdims=True)
        return (v - mu) / jnp.sqrt(var + 1e-5) * w_ + b_

    nt = (((1,), (1,)), ((), ()))
    for i in range(BPP):
        sx = sx_ref[i]
        gx = gx_ref[i]
        dx = gx - sx
        out1_ref[i] = sx + dx
        nq = ln(dx, lnp_ref[0:1, :], lnp_ref[1:2, :])
        nk = ln(sx, lnp_ref[2:3, :], lnp_ref[3:4, :])
        q = lax.dot_general(nq, wq_ref[...], nt, preferred_element_type=jnp.float32)
        k = lax.dot_general(nk, wk_ref[...], nt, preferred_element_type=jnp.float32)
        attn = lax.dot_general(q, k, nt, preferred_element_type=jnp.float32) * 0.0625
        attn = attn - jnp.max(attn, axis=1, keepdims=True)
        attn = jnp.exp(attn)
        attn = attn / jnp.sum(attn, axis=1, keepdims=True)
        c2 = jnp.dot(attn, vpad_ref[i], preferred_element_type=jnp.float32)
        out2_ref[i] = scpad_ref[i] + c2


def kernel(x, coor, Wq, Wk, ln_q_w, ln_q_b, ln_k_w, ln_k_b):
    cx = coor[:, :, 0]
    cy = coor[:, :, 1]
    cz = coor[:, :, 2]

    sflat_o, scxo, scyo, sczo, cm8 = pl.pallas_call(
        _fps_kernel,
        out_shape=[
            jax.ShapeDtypeStruct((B, S), jnp.int32),
            jax.ShapeDtypeStruct((B, S), jnp.float32),
            jax.ShapeDtypeStruct((B, S), jnp.float32),
            jax.ShapeDtypeStruct((B, S), jnp.float32),
            jax.ShapeDtypeStruct((B, 4, N), jnp.float32),
        ],
        scratch_shapes=[pltpu.VMEM((B, N), jnp.float32),
                        pltpu.VMEM((B, S * 128), jnp.float32),
                        pltpu.VMEM((B, S * 128), jnp.float32),
                        pltpu.VMEM((B, S * 128), jnp.float32),
                        pltpu.VMEM((B, S * 128), jnp.float32)],
    )(cx, cy, cz)

    gidxf, coords = pl.pallas_call(
        _bq_kernel,
        grid=(B // BPP,),
        in_specs=[
            pl.BlockSpec((BPP, 1, N), lambda b: (b, 0, 0)),
            pl.BlockSpec((BPP, 1, N), lambda b: (b, 0, 0)),
            pl.BlockSpec((BPP, 1, N), lambda b: (b, 0, 0)),
            pl.BlockSpec((BPP, S, 1), lambda b: (b, 0, 0)),
            pl.BlockSpec((BPP, S, 1), lambda b: (b, 0, 0)),
            pl.BlockSpec((BPP, S, 1), lambda b: (b, 0, 0)),
            pl.BlockSpec((BPP, 4, N), lambda b: (b, 0, 0)),
        ],
        out_specs=[
            pl.BlockSpec((BPP, S, K), lambda b: (b, 0, 0)),
            pl.BlockSpec((BPP, S, 8), lambda b: (b, 0, 0)),
        ],
        out_shape=[
            jax.ShapeDtypeStruct((B, S, K), jnp.int32),
            jax.ShapeDtypeStruct((B, S, 8), jnp.float32),
        ],
        scratch_shapes=[pltpu.VMEM((S, K * 128), jnp.float32),
                        pltpu.VMEM((S, 8 * 128), jnp.float32)],
    )(cx.reshape(B, 1, N), cy.reshape(B, 1, N), cz.reshape(B, 1, N),
      scxo.reshape(B, S, 1), scyo.reshape(B, S, 1), sczo.reshape(B, S, 1),
      cm8)

    gflat = gidxf.reshape(NW, 2, 128)
    sflat = sflat_o.reshape(NW, SPW)
    gx_flat, sx_flat = _sc_gather_call(x.reshape(B * N, C), gflat, sflat)

    sxr = sx_flat.reshape(B, S, C)
    gxr = gx_flat.reshape(B, S, C)
    sample_coor = coords[:, :, 0:3]
    diff_coor = coords[:, :, 3:6]
    v2d = jnp.transpose(diff_coor, (0, 2, 1)).reshape(B, S, 3)
    vpad = jnp.pad(v2d, ((0, 0), (0, 0), (0, 5)))
    scpad = jnp.pad(sample_coor, ((0, 0), (0, 0), (0, 5)))
    lnp = jnp.concatenate([ln_q_w.reshape(1, C), ln_q_b.reshape(1, C),
                           ln_k_w.reshape(1, C), ln_k_b.reshape(1, C)], axis=0)

    out1, out2pad = pl.pallas_call(
        _attn_kernel,
        grid=(B // BPP,),
        in_specs=[
            pl.BlockSpec((BPP, S, C), lambda b: (b, 0, 0)),
            pl.BlockSpec((BPP, S, C), lambda b: (b, 0, 0)),
            pl.BlockSpec((C, C), lambda b: (0, 0)),
            pl.BlockSpec((C, C), lambda b: (0, 0)),
            pl.BlockSpec((4, C), lambda b: (0, 0)),
            pl.BlockSpec((BPP, S, 8), lambda b: (b, 0, 0)),
            pl.BlockSpec((BPP, S, 8), lambda b: (b, 0, 0)),
        ],
        out_specs=[
            pl.BlockSpec((BPP, S, C), lambda b: (b, 0, 0)),
            pl.BlockSpec((BPP, S, 8), lambda b: (b, 0, 0)),
        ],
        out_shape=[
            jax.ShapeDtypeStruct((B, S, C), jnp.float32),
            jax.ShapeDtypeStruct((B, S, 8), jnp.float32),
        ],
    )(sxr, gxr, Wq, Wk, lnp, vpad, scpad)

    return (out1, out2pad[:, :, 0:3])
